# baseline (device time: 92661 ns/iter reference)
import jax
import jax.numpy as jnp
from jax import lax
from jax.experimental import pallas as pl
from jax.experimental.pallas import tpu as pltpu

N_DEV = 4
E_PER_DEV = 4
N_EXPERTS = N_DEV * E_PER_DEV


def kernel(x, router_W, route_idx, expert_W, shared_W):
    n_tok, d_model = x.shape
    e_loc, _, d_ff = expert_W.shape

    ew2 = expert_W.astype(jnp.bfloat16).reshape(2, 2 * d_model, d_ff)
    sw_b = shared_W.astype(jnp.bfloat16)

    def body(x_ref, rw_ref, idx_ref, ew_ref, sw_ref, out_ref,
             commR, commL, sR, rR, sL, rL):
        my = lax.axis_index("i")
        left = lax.rem(my + N_DEV - 1, N_DEV)
        right = lax.rem(my + 1, N_DEV)
        opp = lax.rem(my + 2, N_DEV)

        barrier_sem = pltpu.get_barrier_semaphore()
        for nbr in (left, right):
            pl.semaphore_signal(barrier_sem, inc=1, device_id=(nbr,),
                                device_id_type=pl.DeviceIdType.MESH)
        pl.semaphore_wait(barrier_sem, 2)

        def mk(src, dst, ssem, rsem, dev):
            return pltpu.make_async_remote_copy(
                src_ref=src, dst_ref=dst, send_sem=ssem, recv_sem=rsem,
                device_id=(dev,), device_id_type=pl.DeviceIdType.MESH)

        toR = [mk(ew_ref.at[0], commR.at[0], sR.at[0], rR.at[0], right),
               mk(ew_ref.at[1], commR.at[1], sR.at[1], rR.at[1], right),
               mk(commR.at[0], commR.at[2], sR.at[2], rR.at[2], right)]
        toL = [mk(ew_ref.at[1], commL.at[0], sL.at[0], rL.at[0], left),
               mk(ew_ref.at[0], commL.at[1], sL.at[1], rL.at[1], left),
               mk(commL.at[0], commL.at[2], sL.at[2], rL.at[2], left)]

        toR[0].start(); toR[1].start()
        toL[0].start(); toL[1].start()

        x32 = x_ref[:, :]
        scores = jnp.dot(x32, rw_ref[:, :], preferred_element_type=jnp.float32)
        m = jnp.max(scores, axis=-1, keepdims=True)
        ex = jnp.exp(scores - m)
        probs = ex / jnp.sum(ex, axis=-1, keepdims=True)
        idx = idx_ref[:, :]
        onehot = lax.broadcasted_iota(jnp.int32, (n_tok, N_EXPERTS), 1) == idx
        p_top = jnp.sum(jnp.where(onehot, probs, 0.0), axis=-1, keepdims=True)

        xb = x32.astype(jnp.bfloat16)

        out_ref[:, :] = jnp.dot(xb, sw_ref[:, :],
                                preferred_element_type=jnp.float32)

        def acc_pair(wpair_ref, e_a, e_b):
            pa = jnp.where(idx == e_a, p_top, jnp.float32(0.0)).astype(jnp.bfloat16)
            pb = jnp.where(idx == e_b, p_top, jnp.float32(0.0)).astype(jnp.bfloat16)
            y = jnp.concatenate([pa * xb, pb * xb], axis=1)
            out_ref[:, :] += jnp.dot(y, wpair_ref[:, :],
                                     preferred_element_type=jnp.float32)

        toR[0].wait_recv(); toR[2].start(); acc_pair(commR.at[0], left * 4 + 0, left * 4 + 1)
        toL[0].wait_recv(); toL[2].start(); acc_pair(commL.at[0], right * 4 + 2, right * 4 + 3)

        acc_pair(ew_ref.at[0], my * 4 + 0, my * 4 + 1)
        acc_pair(ew_ref.at[1], my * 4 + 2, my * 4 + 3)

        toR[1].wait_recv(); acc_pair(commR.at[1], left * 4 + 2, left * 4 + 3)
        toL[1].wait_recv(); acc_pair(commL.at[1], right * 4 + 0, right * 4 + 1)
        toR[2].wait_recv(); acc_pair(commR.at[2], opp * 4 + 0, opp * 4 + 1)
        toL[2].wait_recv(); acc_pair(commL.at[2], opp * 4 + 2, opp * 4 + 3)

        for d in toR + toL:
            d.wait_send()

    return pl.pallas_call(
        body,
        out_shape=jax.ShapeDtypeStruct((n_tok, d_ff), jnp.float32),
        in_specs=[pl.BlockSpec(memory_space=pltpu.VMEM)] * 5,
        out_specs=pl.BlockSpec(memory_space=pltpu.VMEM),
        scratch_shapes=[
            pltpu.VMEM((3, 2 * d_model, d_ff), jnp.bfloat16),
            pltpu.VMEM((3, 2 * d_model, d_ff), jnp.bfloat16),
            pltpu.SemaphoreType.DMA((3,)),
            pltpu.SemaphoreType.DMA((3,)),
            pltpu.SemaphoreType.DMA((3,)),
            pltpu.SemaphoreType.DMA((3,)),
        ],
        compiler_params=pltpu.CompilerParams(collective_id=0),
    )(x, router_W, route_idx, ew2, sw_b)
